# baseline (speedup 1.0000x reference)
import functools

import jax
import jax.numpy as jnp
from jax import lax
from jax.experimental import pallas as pl
from jax.experimental.pallas import tpu as pltpu
from jax.experimental.pallas import tpu_sc as plsc

_N = 16384
_CIN = 64
_COUT = 128
_K = 16
_M = _N // 4
_EPS = 1e-5
_TPAD = 128
_BIG = 1 << 30

_QB = 64
_RB = 2048


def _fps_body(px_ref, py_ref, pz_ref, npx_ref, npy_ref, npz_ref):
    riota = lax.broadcasted_iota(jnp.int32, (128, 1), 0)
    liota = lax.broadcasted_iota(jnp.int32, (1, 128), 1)
    lfull = lax.broadcasted_iota(jnp.int32, (128, 128), 1)
    lx0 = px_ref[0, 0]
    ly0 = py_ref[0, 0]
    lz0 = pz_ref[0, 0]
    first = liota == 0
    b0x = jnp.where(first, lx0, 0.0)
    b0y = jnp.where(first, ly0, 0.0)
    b0z = jnp.where(first, lz0, 0.0)
    d0 = jnp.full((128, 128), jnp.inf, jnp.float32)

    def body(i, carry):
        lx, ly, lz, bx, by, bz, dists = carry
        dx = px_ref[:] - lx
        dy = py_ref[:] - ly
        dz = pz_ref[:] - lz
        d = dx * dx + dy * dy + dz * dz
        dm = jnp.minimum(dists, d)
        rmax = jnp.max(dm, axis=1, keepdims=True)
        mx = jnp.max(rmax)
        r = jnp.min(jnp.where(rmax == mx, riota, _BIG))
        cpr = jnp.min(jnp.where(dm == mx, lfull, _BIG), axis=1, keepdims=True)
        c = jnp.sum(jnp.where(riota == r, cpr, 0))
        lmask = liota == c
        nlx = jnp.sum(jnp.where(lmask, px_ref[pl.ds(r, 1), :], 0.0))
        nly = jnp.sum(jnp.where(lmask, py_ref[pl.ds(r, 1), :], 0.0))
        nlz = jnp.sum(jnp.where(lmask, pz_ref[pl.ds(r, 1), :], 0.0))
        here = liota == (i % 128)
        bx = jnp.where(here, nlx, bx)
        by = jnp.where(here, nly, by)
        bz = jnp.where(here, nlz, bz)
        g = i // 128
        npx_ref[pl.ds(g, 1), :] = bx
        npy_ref[pl.ds(g, 1), :] = by
        npz_ref[pl.ds(g, 1), :] = bz
        return (nlx, nly, nlz, bx, by, bz, dm)

    lax.fori_loop(1, _M, body, (lx0, ly0, lz0, b0x, b0y, b0z, d0))


def _fps(px2, py2, pz2, interpret=False):
    return pl.pallas_call(
        _fps_body,
        out_shape=[jax.ShapeDtypeStruct((32, 128), jnp.float32)] * 3,
        interpret=interpret,
    )(px2, py2, pz2)


def _knn_body(qx_ref, qy_ref, qz_ref, px_ref, py_ref, pz_ref, nn_ref, d2_ref):
    lane16 = lax.broadcasted_iota(jnp.int32, (_QB, _K), 1)

    def compute_d2():
        dx = qx_ref[:] - px_ref[:]
        dy = qy_ref[:] - py_ref[:]
        dz = qz_ref[:] - pz_ref[:]
        return dx * dx + dy * dy + dz * dz

    viota = lax.broadcasted_iota(jnp.int32, (_QB, 128, 128), 1)
    j128 = lax.broadcasted_iota(jnp.int32, (_QB, 128), 1)
    vals = []
    gidx = []
    w = compute_d2().reshape(_QB, 128, 128)
    for k in range(4):
        mk = jnp.min(w, axis=1)
        vk = jnp.min(jnp.where(w == mk[:, None, :], viota, _BIG), axis=1)
        masked = jnp.where(viota == vk[:, None, :], jnp.inf, w)
        if k < 3:
            d2_ref[:] = masked.reshape(_QB, _N)
            w = d2_ref[:].reshape(_QB, 128, 128)
        vals.append(mk)
        gidx.append(vk * 128 + j128)

    cur = vals[0]
    curi = gidx[0]
    cnt = jnp.zeros((_QB, 128), jnp.int32)
    nn = jnp.zeros((_QB, _K), jnp.int32)
    for k in range(_K):
        m = jnp.min(cur, axis=1, keepdims=True)
        sel = jnp.min(jnp.where(cur == m, curi, _BIG), axis=1, keepdims=True)
        hit = curi == sel
        cnt = cnt + hit.astype(jnp.int32)
        nv = jnp.where(cnt == 1, vals[1],
                       jnp.where(cnt == 2, vals[2],
                                 jnp.where(cnt == 3, vals[3], jnp.inf)))
        ni = jnp.where(cnt == 1, gidx[1],
                       jnp.where(cnt == 2, gidx[2],
                                 jnp.where(cnt == 3, gidx[3], _BIG)))
        cur = jnp.where(hit, nv, cur)
        curi = jnp.where(hit, ni, curi)
        nn = jnp.where(lane16 == k, sel, nn)

    def full_scan():
        lane = lax.broadcasted_iota(jnp.int32, (_QB, _N), 1)
        d2_ref[:] = compute_d2()

        def step(k, acc):
            d2 = d2_ref[:]
            m = jnp.min(d2, axis=1, keepdims=True)
            sel = jnp.min(jnp.where(d2 == m, lane, _BIG), axis=1, keepdims=True)
            d2_ref[:] = jnp.where(lane == sel, jnp.inf, d2)
            return jnp.where(lane16 == k, sel, acc)

        return lax.fori_loop(0, _K, step, jnp.zeros((_QB, _K), jnp.int32))

    nn_ref[:] = lax.cond(jnp.any(cnt >= 4), full_scan, lambda: nn)


def _knn(qx, qy, qz, pxr, pyr, pzr, interpret=False):
    grid = _M // _QB
    qspec = pl.BlockSpec((_QB, 1), lambda i: (i, 0))
    pspec = pl.BlockSpec((1, _N), lambda i: (0, 0))
    return pl.pallas_call(
        _knn_body,
        grid=(grid,),
        in_specs=[qspec, qspec, qspec, pspec, pspec, pspec],
        out_specs=pl.BlockSpec((_QB, _K), lambda i: (i, 0)),
        out_shape=jax.ShapeDtypeStruct((_M, _K), jnp.int32),
        scratch_shapes=[pltpu.VMEM((_QB, _N), jnp.float32)],
        interpret=interpret,
    )(qx, qy, qz, pxr, pyr, pzr)


def _sc_gather(idx_flat, table):
    info = plsc.get_sparse_core_info()
    nw = info.num_cores * info.num_subcores
    rows_per_w = (_M * _K) // nw
    chunk = 128
    nchunk = rows_per_w // chunk
    mesh = plsc.VectorSubcoreMesh(core_axis_name="c", subcore_axis_name="s")

    @functools.partial(
        pl.kernel,
        mesh=mesh,
        out_type=jax.ShapeDtypeStruct((_M * _K, _TPAD), jnp.float32),
        scratch_types=[
            pltpu.VMEM((chunk,), jnp.int32),
            pltpu.VMEM((chunk, _TPAD), jnp.float32),
            pltpu.SemaphoreType.DMA,
        ],
    )
    def gather_k(idx_hbm, table_hbm, out_hbm, idx_v, rows_v, sem):
        wid = lax.axis_index("s") * info.num_cores + lax.axis_index("c")
        base = wid * rows_per_w
        for j in range(nchunk):
            off = base + j * chunk
            pltpu.sync_copy(idx_hbm.at[pl.ds(off, chunk)], idx_v)
            pltpu.async_copy(table_hbm.at[idx_v], rows_v, sem).wait()
            pltpu.sync_copy(rows_v, out_hbm.at[pl.ds(off, chunk)])

    return gather_k(idx_flat, table)


def _mm_body(g_ref, npx_ref, npy_ref, npz_ref, w3_ref, wp_ref,
             zmax_ref, zmin_ref, s1_ref, s2_ref):
    z = jnp.dot(g_ref[:], wp_ref[:], preferred_element_type=jnp.float32)
    qproj = (npx_ref[:] * w3_ref[0:1, :] + npy_ref[:] * w3_ref[1:2, :]
             + npz_ref[:] * w3_ref[2:3, :])
    zq = z.reshape(_RB // _K, _K, _COUT) - qproj[:, None, :]
    zmax_ref[:] = jnp.max(zq, axis=1)
    zmin_ref[:] = jnp.min(zq, axis=1)

    @pl.when(pl.program_id(0) == 0)
    def _():
        s1_ref[:] = jnp.zeros_like(s1_ref)
        s2_ref[:] = jnp.zeros_like(s2_ref)

    s1_ref[:] += jnp.sum(jnp.sum(zq, axis=1), axis=0, keepdims=True)
    s2_ref[:] += jnp.sum(jnp.sum(zq * zq, axis=1), axis=0, keepdims=True)


def _mm(grouped, npx_c, npy_c, npz_c, w3, wp, interpret=False):
    grid = (_M * _K) // _RB
    qb = _RB // _K
    cspec = pl.BlockSpec((qb, 1), lambda i: (i, 0))
    full = lambda shape: pl.BlockSpec(shape, lambda i: (0, 0))
    return pl.pallas_call(
        _mm_body,
        grid=(grid,),
        in_specs=[
            pl.BlockSpec((_RB, _TPAD), lambda i: (i, 0)),
            cspec, cspec, cspec,
            full((8, _COUT)),
            full((_TPAD, _COUT)),
        ],
        out_specs=[
            pl.BlockSpec((qb, _COUT), lambda i: (i, 0)),
            pl.BlockSpec((qb, _COUT), lambda i: (i, 0)),
            full((1, _COUT)),
            full((1, _COUT)),
        ],
        out_shape=[
            jax.ShapeDtypeStruct((_M, _COUT), jnp.float32),
            jax.ShapeDtypeStruct((_M, _COUT), jnp.float32),
            jax.ShapeDtypeStruct((1, _COUT), jnp.float32),
            jax.ShapeDtypeStruct((1, _COUT), jnp.float32),
        ],
        interpret=interpret,
    )(grouped, npx_c, npy_c, npz_c, w3, wp)


def _fin_body(zmax_ref, zmin_ref, s1_ref, s2_ref, g_ref, b_ref, out_ref):
    cnt = jnp.float32(_M * _K)
    mean = s1_ref[:] / cnt
    var = s2_ref[:] / cnt - mean * mean
    sq = jnp.sqrt(var + _EPS)
    gm = g_ref[:]
    bt = b_ref[:]
    a = (zmax_ref[:] - mean) / sq * gm + bt
    b2 = (zmin_ref[:] - mean) / sq * gm + bt
    out_ref[:] = jnp.maximum(jnp.where(gm > 0, a, b2), 0.0)


def _fin(zmax, zmin, s1, s2, gm, bt, interpret=False):
    return pl.pallas_call(
        _fin_body,
        out_shape=jax.ShapeDtypeStruct((_M, _COUT), jnp.float32),
        interpret=interpret,
    )(zmax, zmin, s1, s2, gm, bt)


def kernel(p, x, o, W, gamma, beta):
    del o
    px2 = p[:, 0].reshape(128, 128)
    py2 = p[:, 1].reshape(128, 128)
    pz2 = p[:, 2].reshape(128, 128)
    npx, npy, npz = _fps(px2, py2, pz2)
    n_p = jnp.stack([npx.reshape(-1), npy.reshape(-1), npz.reshape(-1)], axis=1)

    nn = _knn(
        npx.reshape(_M, 1), npy.reshape(_M, 1), npz.reshape(_M, 1),
        p[:, 0].reshape(1, _N), p[:, 1].reshape(1, _N), p[:, 2].reshape(1, _N),
    )

    table = jnp.concatenate(
        [p, jnp.zeros((_N, 13), jnp.float32), x,
         jnp.zeros((_N, _TPAD - 16 - _CIN), jnp.float32)], axis=1)
    grouped = _sc_gather(nn.reshape(-1), table)

    w3 = jnp.zeros((8, _COUT), jnp.float32).at[0:3, :].set(W[:, :3].T)
    wp = jnp.concatenate(
        [W[:, :3], jnp.zeros((_COUT, 13), jnp.float32), W[:, 3:],
         jnp.zeros((_COUT, _TPAD - 16 - _CIN), jnp.float32)], axis=1).T
    zmax, zmin, s1, s2 = _mm(
        grouped, npx.reshape(_M, 1), npy.reshape(_M, 1), npz.reshape(_M, 1),
        w3, wp)

    x_out = _fin(zmax, zmin, s1, s2,
                 gamma.reshape(1, _COUT), beta.reshape(1, _COUT))
    n_o = jnp.array([_M], dtype=jnp.int32)
    return (n_p, x_out, n_o)


# speedup vs baseline: 1.0785x; 1.0785x over previous
import functools

import jax
import jax.numpy as jnp
from jax import lax
from jax.experimental import pallas as pl
from jax.experimental.pallas import tpu as pltpu
from jax.experimental.pallas import tpu_sc as plsc

_N = 16384
_CIN = 64
_COUT = 128
_K = 16
_M = _N // 4
_EPS = 1e-5
_TPAD = 128
_BIG = 1 << 30

_QB = 64
_RB = 2048


def _fps_body(px_ref, py_ref, pz_ref, npx_ref, npy_ref, npz_ref, dists_ref):
    px = px_ref[:]
    py = py_ref[:]
    pz = pz_ref[:]
    riota = lax.broadcasted_iota(jnp.int32, (128, 1), 0)
    liota = lax.broadcasted_iota(jnp.int32, (1, 128), 1)
    lx0 = px[0:1, 0:1]
    ly0 = py[0:1, 0:1]
    lz0 = pz[0:1, 0:1]
    dists_ref[:] = jnp.full((128, 128), jnp.inf, jnp.float32)
    first = liota == 0
    b0x = jnp.where(first, lx0, 0.0)
    b0y = jnp.where(first, ly0, 0.0)
    b0z = jnp.where(first, lz0, 0.0)

    def body(i, carry):
        lx, ly, lz, bx, by, bz = carry
        dx = px - lx
        dy = py - ly
        dz = pz - lz
        d = dx * dx + dy * dy + dz * dz
        dm = jnp.minimum(dists_ref[:], d)
        dists_ref[:] = dm
        rmax = jnp.max(dm, axis=1, keepdims=True)
        mx = jnp.max(rmax, axis=0, keepdims=True)
        r = jnp.min(jnp.where(rmax == mx, riota, _BIG))
        row = dists_ref[pl.ds(r, 1), :]
        c = jnp.min(jnp.where(row == mx, liota, _BIG), axis=1, keepdims=True)
        lmask = liota == c
        nlx = jnp.sum(jnp.where(lmask, px_ref[pl.ds(r, 1), :], 0.0),
                      axis=1, keepdims=True)
        nly = jnp.sum(jnp.where(lmask, py_ref[pl.ds(r, 1), :], 0.0),
                      axis=1, keepdims=True)
        nlz = jnp.sum(jnp.where(lmask, pz_ref[pl.ds(r, 1), :], 0.0),
                      axis=1, keepdims=True)
        here = liota == (i % 128)
        bx = jnp.where(here, nlx, bx)
        by = jnp.where(here, nly, by)
        bz = jnp.where(here, nlz, bz)
        g = i // 128
        npx_ref[pl.ds(g, 1), :] = bx
        npy_ref[pl.ds(g, 1), :] = by
        npz_ref[pl.ds(g, 1), :] = bz
        return (nlx, nly, nlz, bx, by, bz)

    lax.fori_loop(1, _M, body, (lx0, ly0, lz0, b0x, b0y, b0z))


def _fps(px2, py2, pz2, interpret=False):
    return pl.pallas_call(
        _fps_body,
        out_shape=[jax.ShapeDtypeStruct((32, 128), jnp.float32)] * 3,
        scratch_shapes=[pltpu.VMEM((128, 128), jnp.float32)],
        interpret=interpret,
    )(px2, py2, pz2)


def _knn_body(qx_ref, qy_ref, qz_ref, px_ref, py_ref, pz_ref, nn_ref, d2_ref):
    lane16 = lax.broadcasted_iota(jnp.int32, (_QB, _K), 1)

    def compute_d2():
        dx = qx_ref[:] - px_ref[:]
        dy = qy_ref[:] - py_ref[:]
        dz = qz_ref[:] - pz_ref[:]
        return dx * dx + dy * dy + dz * dz

    viota = lax.broadcasted_iota(jnp.int32, (_QB, 128, 128), 1)
    j128 = lax.broadcasted_iota(jnp.int32, (_QB, 128), 1)
    vals = []
    gidx = []
    w = compute_d2().reshape(_QB, 128, 128)
    for k in range(4):
        mk = jnp.min(w, axis=1)
        vk = jnp.min(jnp.where(w == mk[:, None, :], viota, _BIG), axis=1)
        masked = jnp.where(viota == vk[:, None, :], jnp.inf, w)
        if k < 3:
            d2_ref[:] = masked.reshape(_QB, _N)
            w = d2_ref[:].reshape(_QB, 128, 128)
        vals.append(mk)
        gidx.append(vk * 128 + j128)

    cur = vals[0]
    curi = gidx[0]
    cnt = jnp.zeros((_QB, 128), jnp.int32)
    nn = jnp.zeros((_QB, _K), jnp.int32)
    for k in range(_K):
        m = jnp.min(cur, axis=1, keepdims=True)
        sel = jnp.min(jnp.where(cur == m, curi, _BIG), axis=1, keepdims=True)
        hit = curi == sel
        cnt = cnt + hit.astype(jnp.int32)
        nv = jnp.where(cnt == 1, vals[1],
                       jnp.where(cnt == 2, vals[2],
                                 jnp.where(cnt == 3, vals[3], jnp.inf)))
        ni = jnp.where(cnt == 1, gidx[1],
                       jnp.where(cnt == 2, gidx[2],
                                 jnp.where(cnt == 3, gidx[3], _BIG)))
        cur = jnp.where(hit, nv, cur)
        curi = jnp.where(hit, ni, curi)
        nn = jnp.where(lane16 == k, sel, nn)

    def full_scan():
        lane = lax.broadcasted_iota(jnp.int32, (_QB, _N), 1)
        d2_ref[:] = compute_d2()

        def step(k, acc):
            d2 = d2_ref[:]
            m = jnp.min(d2, axis=1, keepdims=True)
            sel = jnp.min(jnp.where(d2 == m, lane, _BIG), axis=1, keepdims=True)
            d2_ref[:] = jnp.where(lane == sel, jnp.inf, d2)
            return jnp.where(lane16 == k, sel, acc)

        return lax.fori_loop(0, _K, step, jnp.zeros((_QB, _K), jnp.int32))

    nn_ref[:] = lax.cond(jnp.any(cnt >= 4), full_scan, lambda: nn)


def _knn(qx, qy, qz, pxr, pyr, pzr, interpret=False):
    grid = _M // _QB
    qspec = pl.BlockSpec((_QB, 1), lambda i: (i, 0))
    pspec = pl.BlockSpec((1, _N), lambda i: (0, 0))
    return pl.pallas_call(
        _knn_body,
        grid=(grid,),
        in_specs=[qspec, qspec, qspec, pspec, pspec, pspec],
        out_specs=pl.BlockSpec((_QB, _K), lambda i: (i, 0)),
        out_shape=jax.ShapeDtypeStruct((_M, _K), jnp.int32),
        scratch_shapes=[pltpu.VMEM((_QB, _N), jnp.float32)],
        interpret=interpret,
    )(qx, qy, qz, pxr, pyr, pzr)


def _sc_gather(idx_flat, table):
    info = plsc.get_sparse_core_info()
    nw = info.num_cores * info.num_subcores
    rows_per_w = (_M * _K) // nw
    chunk = 128
    nchunk = rows_per_w // chunk
    mesh = plsc.VectorSubcoreMesh(core_axis_name="c", subcore_axis_name="s")

    @functools.partial(
        pl.kernel,
        mesh=mesh,
        out_type=jax.ShapeDtypeStruct((_M * _K, _TPAD), jnp.float32),
        scratch_types=[
            pltpu.VMEM((chunk,), jnp.int32),
            pltpu.VMEM((chunk, _TPAD), jnp.float32),
            pltpu.SemaphoreType.DMA,
        ],
    )
    def gather_k(idx_hbm, table_hbm, out_hbm, idx_v, rows_v, sem):
        wid = lax.axis_index("s") * info.num_cores + lax.axis_index("c")
        base = wid * rows_per_w
        for j in range(nchunk):
            off = base + j * chunk
            pltpu.sync_copy(idx_hbm.at[pl.ds(off, chunk)], idx_v)
            pltpu.async_copy(table_hbm.at[idx_v], rows_v, sem).wait()
            pltpu.sync_copy(rows_v, out_hbm.at[pl.ds(off, chunk)])

    return gather_k(idx_flat, table)


def _mm_body(g_ref, npx_ref, npy_ref, npz_ref, w3_ref, wp_ref,
             zmax_ref, zmin_ref, s1_ref, s2_ref):
    z = jnp.dot(g_ref[:], wp_ref[:], preferred_element_type=jnp.float32)
    qproj = (npx_ref[:] * w3_ref[0:1, :] + npy_ref[:] * w3_ref[1:2, :]
             + npz_ref[:] * w3_ref[2:3, :])
    zq = z.reshape(_RB // _K, _K, _COUT) - qproj[:, None, :]
    zmax_ref[:] = jnp.max(zq, axis=1)
    zmin_ref[:] = jnp.min(zq, axis=1)

    @pl.when(pl.program_id(0) == 0)
    def _():
        s1_ref[:] = jnp.zeros_like(s1_ref)
        s2_ref[:] = jnp.zeros_like(s2_ref)

    s1_ref[:] += jnp.sum(jnp.sum(zq, axis=1), axis=0, keepdims=True)
    s2_ref[:] += jnp.sum(jnp.sum(zq * zq, axis=1), axis=0, keepdims=True)


def _mm(grouped, npx_c, npy_c, npz_c, w3, wp, interpret=False):
    grid = (_M * _K) // _RB
    qb = _RB // _K
    cspec = pl.BlockSpec((qb, 1), lambda i: (i, 0))
    full = lambda shape: pl.BlockSpec(shape, lambda i: (0, 0))
    return pl.pallas_call(
        _mm_body,
        grid=(grid,),
        in_specs=[
            pl.BlockSpec((_RB, _TPAD), lambda i: (i, 0)),
            cspec, cspec, cspec,
            full((8, _COUT)),
            full((_TPAD, _COUT)),
        ],
        out_specs=[
            pl.BlockSpec((qb, _COUT), lambda i: (i, 0)),
            pl.BlockSpec((qb, _COUT), lambda i: (i, 0)),
            full((1, _COUT)),
            full((1, _COUT)),
        ],
        out_shape=[
            jax.ShapeDtypeStruct((_M, _COUT), jnp.float32),
            jax.ShapeDtypeStruct((_M, _COUT), jnp.float32),
            jax.ShapeDtypeStruct((1, _COUT), jnp.float32),
            jax.ShapeDtypeStruct((1, _COUT), jnp.float32),
        ],
        interpret=interpret,
    )(grouped, npx_c, npy_c, npz_c, w3, wp)


def _fin_body(zmax_ref, zmin_ref, s1_ref, s2_ref, g_ref, b_ref, out_ref):
    cnt = jnp.float32(_M * _K)
    mean = s1_ref[:] / cnt
    var = s2_ref[:] / cnt - mean * mean
    sq = jnp.sqrt(var + _EPS)
    gm = g_ref[:]
    bt = b_ref[:]
    a = (zmax_ref[:] - mean) / sq * gm + bt
    b2 = (zmin_ref[:] - mean) / sq * gm + bt
    out_ref[:] = jnp.maximum(jnp.where(gm > 0, a, b2), 0.0)


def _fin(zmax, zmin, s1, s2, gm, bt, interpret=False):
    return pl.pallas_call(
        _fin_body,
        out_shape=jax.ShapeDtypeStruct((_M, _COUT), jnp.float32),
        interpret=interpret,
    )(zmax, zmin, s1, s2, gm, bt)


def kernel(p, x, o, W, gamma, beta):
    del o
    px2 = p[:, 0].reshape(128, 128)
    py2 = p[:, 1].reshape(128, 128)
    pz2 = p[:, 2].reshape(128, 128)
    npx, npy, npz = _fps(px2, py2, pz2)
    n_p = jnp.stack([npx.reshape(-1), npy.reshape(-1), npz.reshape(-1)], axis=1)

    nn = _knn(
        npx.reshape(_M, 1), npy.reshape(_M, 1), npz.reshape(_M, 1),
        p[:, 0].reshape(1, _N), p[:, 1].reshape(1, _N), p[:, 2].reshape(1, _N),
    )

    table = jnp.concatenate(
        [p, jnp.zeros((_N, 13), jnp.float32), x,
         jnp.zeros((_N, _TPAD - 16 - _CIN), jnp.float32)], axis=1)
    grouped = _sc_gather(nn.reshape(-1), table)

    w3 = jnp.zeros((8, _COUT), jnp.float32).at[0:3, :].set(W[:, :3].T)
    wp = jnp.concatenate(
        [W[:, :3], jnp.zeros((_COUT, 13), jnp.float32), W[:, 3:],
         jnp.zeros((_COUT, _TPAD - 16 - _CIN), jnp.float32)], axis=1).T
    zmax, zmin, s1, s2 = _mm(
        grouped, npx.reshape(_M, 1), npy.reshape(_M, 1), npz.reshape(_M, 1),
        w3, wp)

    x_out = _fin(zmax, zmin, s1, s2,
                 gamma.reshape(1, _COUT), beta.reshape(1, _COUT))
    n_o = jnp.array([_M], dtype=jnp.int32)
    return (n_p, x_out, n_o)
